# Initial kernel scaffold; baseline (speedup 1.0000x reference)
import jax
import jax.numpy as jnp
from jax.experimental import pallas as pl


def kernel(xyz, point_feature):
    raise NotImplementedError("write your pallas kernel here")


# speedup vs baseline: 10.0721x; 10.0721x over previous
import functools

import jax
import jax.numpy as jnp
from jax import lax
from jax.experimental import pallas as pl
from jax.experimental.pallas import tpu as pltpu
from jax.experimental.pallas import tpu_sc as plsc

B = 4
N = 16384
S = 1024
K = 32
C = 64
ROWS = 128
COLS = 128
CR = 8
CC = 128
PAD3 = 64
TW = PAD3 + C


def _fps_body(x_ref, y_ref, z_ref, idx_out, cx_out, cy_out, cz_out,
              dist_ref, fscr, cscr):
    t = pl.program_id(0)

    @pl.when(t == 0)
    def _init():
        dist_ref[...] = jnp.full((B, ROWS, COLS), 1e10, jnp.float32)
        fscr[...] = jnp.zeros((1, B), jnp.int32)
        for b in range(B):
            cscr[0:1, b:b + 1] = x_ref[b, 0:1, 0:1]
            cscr[1:2, b:b + 1] = y_ref[b, 0:1, 0:1]
            cscr[2:3, b:b + 1] = z_ref[b, 0:1, 0:1]

    idx_out[...] = fscr[...].reshape(1, 1, B)
    cx_out[...] = cscr[0:1, :].reshape(1, 1, B)
    cy_out[...] = cscr[1:2, :].reshape(1, 1, B)
    cz_out[...] = cscr[2:3, :].reshape(1, 1, B)

    iota2 = (lax.broadcasted_iota(jnp.int32, (ROWS, COLS), 0) * COLS
             + lax.broadcasted_iota(jnp.int32, (ROWS, COLS), 1))
    for b in range(B):
        xb = x_ref[b]
        yb = y_ref[b]
        zb = z_ref[b]
        dx = xb - cscr[0:1, b:b + 1]
        dy = yb - cscr[1:2, b:b + 1]
        dz = zb - cscr[2:3, b:b + 1]
        d = (dx * dx + dy * dy) + dz * dz
        nd = jnp.minimum(dist_ref[b], d)
        dist_ref[b] = nd
        m = jnp.max(nd, axis=(0, 1), keepdims=True)
        cand = jnp.where(nd == m, iota2, jnp.int32(1 << 30))
        fidx = jnp.min(cand, axis=(0, 1), keepdims=True)
        ex = iota2 == fidx
        fscr[0:1, b:b + 1] = fidx
        zero = jnp.float32(0)
        cscr[0:1, b:b + 1] = jnp.sum(jnp.where(ex, xb, zero), axis=(0, 1),
                                     keepdims=True)
        cscr[1:2, b:b + 1] = jnp.sum(jnp.where(ex, yb, zero), axis=(0, 1),
                                     keepdims=True)
        cscr[2:3, b:b + 1] = jnp.sum(jnp.where(ex, zb, zero), axis=(0, 1),
                                     keepdims=True)


def _fps(xs, ys, zs):
    spec_in = pl.BlockSpec((B, ROWS, COLS), lambda t: (0, 0, 0))
    spec_out = pl.BlockSpec((1, 1, B), lambda t: (t, 0, 0))
    return pl.pallas_call(
        _fps_body,
        grid=(S,),
        in_specs=[spec_in] * 3,
        out_specs=[spec_out] * 4,
        out_shape=[
            jax.ShapeDtypeStruct((S, 1, B), jnp.int32),
            jax.ShapeDtypeStruct((S, 1, B), jnp.float32),
            jax.ShapeDtypeStruct((S, 1, B), jnp.float32),
            jax.ShapeDtypeStruct((S, 1, B), jnp.float32),
        ],
        scratch_shapes=[
            pltpu.VMEM((B, ROWS, COLS), jnp.float32),
            pltpu.VMEM((1, B), jnp.int32),
            pltpu.VMEM((3, B), jnp.float32),
        ],
    )(xs, ys, zs)


_NCHUNK = N // 128

def _knn_body(p_ref, pc_ref, ct_ref, cx_ref, cy_ref, cz_ref, nidx_ref,
              v_ref, i_ref, d_ref):
    b = pl.program_id(0)
    c = pl.program_id(1)

    @pl.when(c == 0)
    def _init():
        for j in range(K):
            v_ref[j] = jnp.full((CR, CC), 3e38, jnp.float32)
            i_ref[j] = jnp.zeros((CR, CC), jnp.int32)

    cxv = cx_ref[0]
    cyv = cy_ref[0]
    czv = cz_ref[0]
    c2 = (cxv * cxv + cyv * cyv) + czv * czv
    pb = p_ref[0].astype(jnp.bfloat16)
    ctb = ct_ref[0].astype(jnp.bfloat16)
    pc = pc_ref[0, 0]
    px = pc[:, 0:1]
    py = pc[:, 1:2]
    pz = pc[:, 2:3]
    p2 = (px * px + py * py) + pz * pz
    for g in range(CR):
        dg = jnp.dot(pb, ctb[:, g * CC:(g + 1) * CC],
                     preferred_element_type=jnp.float32)
        dm = jnp.float32(-2.0) * dg + c2[g:g + 1, :]
        d_ref[:, g:g + 1, :] = (dm + p2).reshape(128, 1, CC)

    base = c * 128

    def group(g8, carry):
        cands = []
        for p in range(8):
            dp = d_ref[g8 * 8 + p]
            ci = jnp.full((CR, CC), base + g8 * 8 + p, jnp.int32)
            cands.append([dp, ci])
        for j in range(K):
            vj = v_ref[j]
            ij = i_ref[j]
            for p in range(8):
                cd, ci = cands[p]
                lt = cd < vj
                nvj = jnp.where(lt, cd, vj)
                nij = jnp.where(lt, ci, ij)
                cands[p][0] = jnp.where(lt, vj, cd)
                cands[p][1] = jnp.where(lt, ij, ci)
                vj = nvj
                ij = nij
            v_ref[j] = vj
            i_ref[j] = ij
        return carry

    lax.fori_loop(0, 16, group, 0)

    @pl.when(c == _NCHUNK - 1)
    def _emit():
        offs = b * N
        for j in range(K):
            nidx_ref[0, j] = i_ref[j] + offs


def _knn(ppad, pxyz, ctpad, cxs, cys, czs):
    return pl.pallas_call(
        _knn_body,
        grid=(B, _NCHUNK),
        in_specs=[
            pl.BlockSpec((1, 128, 128), lambda b, c: (b, c, 0)),
            pl.BlockSpec((1, 1, 128, 3), lambda b, c: (b, c, 0, 0)),
            pl.BlockSpec((1, 128, S), lambda b, c: (b, 0, 0)),
            pl.BlockSpec((1, CR, CC), lambda b, c: (b, 0, 0)),
            pl.BlockSpec((1, CR, CC), lambda b, c: (b, 0, 0)),
            pl.BlockSpec((1, CR, CC), lambda b, c: (b, 0, 0)),
        ],
        out_specs=pl.BlockSpec((1, K, CR, CC), lambda b, c: (b, 0, 0, 0)),
        out_shape=jax.ShapeDtypeStruct((B, K, CR, CC), jnp.int32),
        scratch_shapes=[
            pltpu.VMEM((K, CR, CC), jnp.float32),
            pltpu.VMEM((K, CR, CC), jnp.int32),
            pltpu.VMEM((128, CR, CC), jnp.float32),
        ],
    )(ppad, pxyz, ctpad, cxs, cys, czs)


_NIDX = B * S * (K + 1)
_NW = 32
_PER_W = _NIDX // _NW
_CH = 528
_NCH = _PER_W // _CH


def _sc_gather(table, idx):
    mesh = plsc.VectorSubcoreMesh(core_axis_name="c", subcore_axis_name="s")

    @functools.partial(
        pl.kernel,
        out_type=jax.ShapeDtypeStruct((_NIDX, TW), jnp.float32),
        mesh=mesh,
        scratch_types=[
            pltpu.VMEM((_CH,), jnp.int32),
            pltpu.VMEM((_CH, TW), jnp.float32),
            pltpu.SemaphoreType.DMA,
        ],
    )
    def gather_kernel(table_hbm, idx_hbm, out_hbm, idx_v, rows_v, sem):
        wid = lax.axis_index("s") * 2 + lax.axis_index("c")
        base = wid * _PER_W

        def body(j, carry):
            off = base + j * _CH
            pltpu.sync_copy(idx_hbm.at[pl.ds(off, _CH)], idx_v)
            pltpu.async_copy(table_hbm.at[idx_v], rows_v, sem).wait()
            pltpu.sync_copy(rows_v, out_hbm.at[pl.ds(off, _CH)])
            return carry

        lax.fori_loop(0, _NCH, body, 0)

    return gather_kernel(table, idx)


def kernel(xyz, point_feature):
    xyz = xyz.astype(jnp.float32)
    point_feature = point_feature.astype(jnp.float32)

    xs = xyz[..., 0].reshape(B, ROWS, COLS)
    ys = xyz[..., 1].reshape(B, ROWS, COLS)
    zs = xyz[..., 2].reshape(B, ROWS, COLS)

    fps_idx, cxo, cyo, czo = _fps(xs, ys, zs)
    cxt = cxo.reshape(S, B).T
    cyt = cyo.reshape(S, B).T
    czt = czo.reshape(S, B).T
    center_xyz = jnp.stack([cxt, cyt, czt], axis=-1)

    ppad = jnp.pad(xyz, ((0, 0), (0, 0), (0, 125)))
    ctpad = jnp.pad(jnp.stack([cxt, cyt, czt], axis=1),
                    ((0, 0), (0, 125), (0, 0)))
    nidx = _knn(
        ppad,
        xyz.reshape(B, N // 128, 128, 3),
        ctpad,
        cxt.reshape(B, CR, CC),
        cyt.reshape(B, CR, CC),
        czt.reshape(B, CR, CC),
    )

    neighbor_flat = nidx.transpose(0, 2, 3, 1).reshape(-1)
    fps_flat = (fps_idx.reshape(S, B).T
                + jnp.arange(B, dtype=jnp.int32)[:, None] * N).reshape(-1)
    all_idx = jnp.concatenate([fps_flat, neighbor_flat]).astype(jnp.int32)

    table = jnp.concatenate(
        [jnp.pad(xyz, ((0, 0), (0, 0), (0, PAD3 - 3))), point_feature],
        axis=-1).reshape(B * N, TW)

    rows = _sc_gather(table, all_idx)

    center_feature = rows[:B * S, PAD3:].reshape(B, S, C)
    nb = rows[B * S:]
    neighbor_xyz = nb[:, :3].reshape(B, S, K, 3)
    neighbor_feature = nb[:, PAD3:].reshape(B, S, K, C)
    return (center_xyz, center_feature, neighbor_xyz, neighbor_feature)
